# baseline (speedup 1.0000x reference)
import functools

import jax
import jax.numpy as jnp
from jax import lax
from jax.experimental import pallas as pl
from jax.experimental.pallas import tpu as pltpu
from jax.experimental.pallas import tpu_sc as plsc

N = 10000
E = 320000
D = 128
K = 3
L = 10
OUT = 10
EPS = 1e-5

NC = 2
NS = 16
CH = 64
EP = 327680
EW = EP // (NC * NS)
HCH = EW // CH
NBLK = HCH // 8
NP = 10240
DSEG = NP // NS
ZPAD = 128
GP = N + ZPAD
PR = 10112
PSEG = PR // NS
SEC = 32
NSEC = HCH // SEC

_sc_mesh = plsc.VectorSubcoreMesh(core_axis_name="c", subcore_axis_name="s")


@functools.partial(
    pl.kernel,
    name="sc_prep",
    out_type=(
        jax.ShapeDtypeStruct((NC * NP,), jnp.float32),
        jax.ShapeDtypeStruct((NC, NS, HCH, CH), jnp.int32),
    ),
    mesh=_sc_mesh,
    scratch_types=[
        pltpu.VMEM((8, CH), jnp.int32),
        pltpu.VMEM((8, CH), jnp.int32),
        pltpu.VMEM((8, CH), jnp.float32),
        pltpu.VMEM((DSEG,), jnp.float32),
        pltpu.VMEM_SHARED((NP,), jnp.float32),
    ],
)
def _sc_prep(src_hbm, dst_hbm, deg_out, srcp_out, src_v, dst_v, w_v, z_v,
             deg_acc):
    c = lax.axis_index("c")
    s = lax.axis_index("s")

    def zero16(i, _):
        z_v[pl.ds(i * 16, 16)] = jnp.zeros((16,), jnp.float32)
        return 0
    lax.fori_loop(0, DSEG // 16, zero16, 0)
    pltpu.sync_copy(z_v, deg_acc.at[pl.ds(s * DSEG, DSEG)])
    plsc.subcore_barrier()

    def block(ib, _):
        blk = pl.ds(ib * 8, 8)
        pltpu.sync_copy(src_hbm.at[c, s, blk], src_v)
        pltpu.sync_copy(dst_hbm.at[c, s, blk], dst_v)
        for i in range(8):
            for j in range(CH // 16):
                sl = pl.ds(j * 16, 16)
                s16 = src_v[i, sl]
                d16 = dst_v[i, sl]
                eq = s16 == d16
                w_v[i, sl] = jnp.where(eq, 0.0, 1.0).astype(jnp.float32)
                src_v[i, sl] = jnp.where(
                    eq, N + jnp.bitwise_and(s16, ZPAD - 1), s16)
        for i in range(8):
            pltpu.sync_copy(w_v.at[i], deg_acc.at[src_v.at[i]], add=True)
        pltpu.sync_copy(src_v, srcp_out.at[c, s, blk])
        return 0
    lax.fori_loop(0, NBLK, block, 0)

    plsc.subcore_barrier()
    pltpu.sync_copy(deg_acc.at[pl.ds(s * DSEG, DSEG)],
                    deg_out.at[pl.ds(c * NP + s * DSEG, DSEG)])


@functools.partial(
    pl.kernel,
    name="sc_prop",
    out_type=jax.ShapeDtypeStruct((NC, PR, D), jnp.float32),
    mesh=_sc_mesh,
    scratch_types=[
        pltpu.VMEM((SEC, CH), jnp.int32),
        pltpu.VMEM((SEC, CH), jnp.int32),
        pltpu.VMEM((SEC, CH), jnp.int32),
        pltpu.VMEM((SEC, CH), jnp.int32),
        pltpu.VMEM((CH, D), jnp.float32),
        pltpu.VMEM((CH, D), jnp.float32),
        pltpu.VMEM_SHARED((PR, D), jnp.float32),
        pltpu.SemaphoreType.DMA,
        pltpu.SemaphoreType.DMA,
        pltpu.SemaphoreType.DMA,
        pltpu.SemaphoreType.DMA,
    ],
)
def _sc_prop(g_hbm, srcp_hbm, dst_hbm, p_out,
             sp0, sp1, sd0, sd1, rows0, rows1, acc, sem0, sem1,
             isem0, isem1):
    c = lax.axis_index("c")
    s = lax.axis_index("s")

    def zrow_init(i, _):
        for j in range(D // 16):
            rows0[i, pl.ds(j * 16, 16)] = jnp.zeros((16,), jnp.float32)
        return 0
    lax.fori_loop(0, CH, zrow_init, 0)
    for t in range(PSEG // CH):
        pltpu.sync_copy(rows0, acc.at[pl.ds(s * PSEG + t * CH, CH)])
    pltpu.sync_copy(
        rows0.at[pl.ds(0, PSEG % CH)],
        acc.at[pl.ds(s * PSEG + (PSEG // CH) * CH, PSEG % CH)])

    sidx = ((sp0, sd0), (sp1, sd1))
    isems = (isem0, isem1)
    pltpu.async_copy(srcp_hbm.at[c, s, pl.ds(0, SEC)], sp0, isem0)
    pltpu.async_copy(dst_hbm.at[c, s, pl.ds(0, SEC)], sd0, isem0)
    plsc.subcore_barrier()

    bufs = (rows0, rows1)
    sems = (sem0, sem1)

    for sec in range(NSEC):
        par = sec % 2
        spb, sdb = sidx[par]
        sl = pl.ds(sec * SEC, SEC)
        pltpu.make_async_copy(srcp_hbm.at[c, s, sl], spb, isems[par]).wait()
        pltpu.make_async_copy(dst_hbm.at[c, s, sl], sdb, isems[par]).wait()
        if sec + 1 < NSEC:
            sl2 = pl.ds((sec + 1) * SEC, SEC)
            pltpu.async_copy(srcp_hbm.at[c, s, sl2], sidx[1 - par][0],
                             isems[1 - par])
            pltpu.async_copy(dst_hbm.at[c, s, sl2], sidx[1 - par][1],
                             isems[1 - par])
        pltpu.async_copy(g_hbm.at[spb.at[0]], rows0, sem0)

        def chunk(j, _):
            cur = lax.rem(j, 2)
            for b in range(2):
                @pl.when(cur == b)
                def _():
                    @pl.when(j + 1 < SEC)
                    def _():
                        pltpu.async_copy(g_hbm.at[spb.at[j + 1]],
                                         bufs[1 - b], sems[1 - b])
                    pltpu.make_async_copy(g_hbm.at[spb.at[j]], bufs[b],
                                          sems[b]).wait()
                    pltpu.sync_copy(bufs[b], acc.at[sdb.at[j]], add=True)
            return 0
        lax.fori_loop(0, SEC, chunk, 0)

    plsc.subcore_barrier()
    pltpu.sync_copy(acc.at[pl.ds(s * PSEG, PSEG)],
                    p_out.at[c, pl.ds(s * PSEG, PSEG)])


def _tc_call(body, out_shapes):
    return pl.pallas_call(
        body,
        out_shape=[jax.ShapeDtypeStruct(s, jnp.float32) for s in out_shapes],
    )


def _assemble(p_ref):
    return p_ref[0, :N, :] + p_ref[1, :N, :]


def _write_g(g_ref, gbody):
    g_ref[:N, :] = gbody
    g_ref[N:, :] = jnp.zeros((GP - N, D), jnp.float32)


def _prep_body(d0_ref, d1_ref, x_ref, dis_ref, g_ref):
    deg = d0_ref[...] + d1_ref[...]
    dis = jnp.where(deg > 0, lax.rsqrt(jnp.where(deg > 0, deg, 1.0)), 0.0)
    dis_ref[...] = dis
    _write_g(g_ref, dis * x_ref[...])


def _tcag_body(p_ref, dis_ref, g1_ref):
    dis = dis_ref[...]
    _write_g(g1_ref, -(dis * dis * _assemble(p_ref)))


def _tcmm_body(tx0_ref, p_ref, dis_ref, w0_ref, w1_ref, oacc_ref):
    dis = dis_ref[...]
    tx1 = -(dis * _assemble(p_ref))
    oacc_ref[...] = (
        jnp.dot(tx0_ref[...], w0_ref[...], preferred_element_type=jnp.float32)
        + jnp.dot(tx1, w1_ref[...], preferred_element_type=jnp.float32))


def _tcb_body(tx0_ref, oacc_ref, q_ref, dis_ref, w2_ref, b_ref,
              ginv_ref, beta_ref, h_ref, gn_ref):
    dis = dis_ref[...]
    tx2 = -2.0 * (dis * _assemble(q_ref)) - tx0_ref[...]
    out = (oacc_ref[...]
           + jnp.dot(tx2, w2_ref[...], preferred_element_type=jnp.float32)
           + b_ref[...])
    h = jnp.maximum(out, 0.0) * ginv_ref[...] + beta_ref[...]
    h_ref[...] = h
    _write_g(gn_ref, dis * h)


def _final_body(h_ref, lw_ref, lb_ref, o_ref):
    o_ref[...] = jnp.dot(h_ref[...], lw_ref[...],
                         preferred_element_type=jnp.float32) + lb_ref[...]


def kernel(x, edge_index, W, b, gamma, beta, lin_W, lin_b):
    pad = (jnp.arange(EP - E, dtype=jnp.int32) * 13) % N
    src4 = jnp.concatenate([edge_index[0], pad]).reshape(NC, NS, HCH, CH)
    dst4 = jnp.concatenate([edge_index[1], pad]).reshape(NC, NS, HCH, CH)

    deg_flat, srcp4 = _sc_prep(src4, dst4)

    d0 = deg_flat[:N][:, None]
    d1 = deg_flat[NP:NP + N][:, None]
    dis_col, g = _tc_call(_prep_body, [(N, 1), (GP, D)])(d0, d1, x)

    ginv = gamma * (1.0 / jnp.sqrt(1.0 + EPS))
    lw_pad = jnp.zeros((D, D), jnp.float32).at[:, :OUT].set(lin_W)
    lb_pad = jnp.zeros((1, D), jnp.float32).at[0, :OUT].set(lin_b)

    rep = lambda a: jnp.repeat(a, 2, axis=0)
    w0s, w1s, w2s = rep(W[:, 0]), rep(W[:, 1]), rep(W[:, 2])
    brows = rep(b[:, None, :])
    ginvrows = rep(ginv[:, None, :])
    betarows = rep(beta[:, None, :])
    flags = jnp.tile(jnp.array([1, 0], jnp.int32), L)

    def halfstep(carry, ws):
        h, g, pprev = carry
        w0, w1, w2, brow, ginvrow, betarow, flag = ws
        p = _sc_prop(g, srcp4, dst4)

        def fa(h, g, pprev, p):
            g1, = _tc_call(_tcag_body, [(GP, D)])(p, dis_col)
            return h, g1, p

        def fb(h, g, pprev, p):
            oacc, = _tc_call(_tcmm_body, [(N, D)])(h, pprev, dis_col, w0, w1)
            h1, g1 = _tc_call(_tcb_body, [(N, D), (GP, D)])(
                h, oacc, p, dis_col, w2, brow, ginvrow, betarow)
            return h1, g1, p

        h, g, pprev = lax.cond(flag == 1, fa, fb, h, g, pprev, p)
        return (h, g, pprev), None

    (h, _, _), _ = lax.scan(
        halfstep, (x, g, jnp.zeros((NC, PR, D), jnp.float32)),
        (w0s, w1s, w2s, brows, ginvrows, betarows, flags))

    logits_pad, = _tc_call(_final_body, [(N, D)])(h, lw_pad, lb_pad)
    return logits_pad[:, :OUT]


# speedup vs baseline: 1.0955x; 1.0955x over previous
import functools

import jax
import jax.numpy as jnp
from jax import lax
from jax.experimental import pallas as pl
from jax.experimental.pallas import tpu as pltpu
from jax.experimental.pallas import tpu_sc as plsc

N = 10000
E = 320000
D = 128
K = 3
L = 10
OUT = 10
EPS = 1e-5

NC = 2
NS = 16
CH = 64
EP = 327680
EW = EP // (NC * NS)
HCH = EW // CH
NBLK = HCH // 8
NP = 10240
DSEG = NP // NS
ZPAD = 128
GP = N + ZPAD
PR = 10112
PSEG = PR // NS
SEC = 32
NSEC = HCH // SEC

_sc_mesh = plsc.VectorSubcoreMesh(core_axis_name="c", subcore_axis_name="s")


@functools.partial(
    pl.kernel,
    name="sc_prep",
    out_type=(
        jax.ShapeDtypeStruct((NC * NP,), jnp.float32),
        jax.ShapeDtypeStruct((NC, NS, HCH, CH), jnp.int32),
    ),
    mesh=_sc_mesh,
    scratch_types=[
        pltpu.VMEM((8, CH), jnp.int32),
        pltpu.VMEM((8, CH), jnp.int32),
        pltpu.VMEM((8, CH), jnp.float32),
        pltpu.VMEM((DSEG,), jnp.float32),
        pltpu.VMEM_SHARED((NP,), jnp.float32),
    ],
)
def _sc_prep(src_hbm, dst_hbm, deg_out, srcp_out, src_v, dst_v, w_v, z_v,
             deg_acc):
    c = lax.axis_index("c")
    s = lax.axis_index("s")

    def zero16(i, _):
        z_v[pl.ds(i * 16, 16)] = jnp.zeros((16,), jnp.float32)
        return 0
    lax.fori_loop(0, DSEG // 16, zero16, 0)
    pltpu.sync_copy(z_v, deg_acc.at[pl.ds(s * DSEG, DSEG)])
    plsc.subcore_barrier()

    def block(ib, _):
        blk = pl.ds(ib * 8, 8)
        pltpu.sync_copy(src_hbm.at[c, s, blk], src_v)
        pltpu.sync_copy(dst_hbm.at[c, s, blk], dst_v)
        for i in range(8):
            for j in range(CH // 16):
                sl = pl.ds(j * 16, 16)
                s16 = src_v[i, sl]
                d16 = dst_v[i, sl]
                eq = s16 == d16
                w_v[i, sl] = jnp.where(eq, 0.0, 1.0).astype(jnp.float32)
                src_v[i, sl] = jnp.where(
                    eq, N + jnp.bitwise_and(s16, ZPAD - 1), s16)
        for i in range(8):
            pltpu.sync_copy(w_v.at[i], deg_acc.at[src_v.at[i]], add=True)
        pltpu.sync_copy(src_v, srcp_out.at[c, s, blk])
        return 0
    lax.fori_loop(0, NBLK, block, 0)

    plsc.subcore_barrier()
    pltpu.sync_copy(deg_acc.at[pl.ds(s * DSEG, DSEG)],
                    deg_out.at[pl.ds(c * NP + s * DSEG, DSEG)])


@functools.partial(
    pl.kernel,
    name="sc_prop",
    out_type=jax.ShapeDtypeStruct((NC, PR, D), jnp.float32),
    mesh=_sc_mesh,
    scratch_types=[
        pltpu.VMEM((SEC, CH), jnp.int32),
        pltpu.VMEM((SEC, CH), jnp.int32),
        pltpu.VMEM((SEC, CH), jnp.int32),
        pltpu.VMEM((SEC, CH), jnp.int32),
        pltpu.VMEM((CH, D), jnp.float32),
        pltpu.VMEM((CH, D), jnp.float32),
        pltpu.VMEM_SHARED((PR, D), jnp.float32),
        pltpu.SemaphoreType.DMA,
        pltpu.SemaphoreType.DMA,
        pltpu.SemaphoreType.DMA,
        pltpu.SemaphoreType.DMA,
    ],
)
def _sc_prop(g_hbm, srcp_hbm, dst_hbm, p_out,
             sp0, sp1, sd0, sd1, rows0, rows1, acc, sem0, sem1,
             isem0, isem1):
    c = lax.axis_index("c")
    s = lax.axis_index("s")

    def zrow_init(i, _):
        for j in range(D // 16):
            rows0[i, pl.ds(j * 16, 16)] = jnp.zeros((16,), jnp.float32)
        return 0
    lax.fori_loop(0, CH, zrow_init, 0)
    for t in range(PSEG // CH):
        pltpu.sync_copy(rows0, acc.at[pl.ds(s * PSEG + t * CH, CH)])
    pltpu.sync_copy(
        rows0.at[pl.ds(0, PSEG % CH)],
        acc.at[pl.ds(s * PSEG + (PSEG // CH) * CH, PSEG % CH)])

    sidx = ((sp0, sd0), (sp1, sd1))
    isems = (isem0, isem1)
    pltpu.async_copy(srcp_hbm.at[c, s, pl.ds(0, SEC)], sp0, isem0)
    pltpu.async_copy(dst_hbm.at[c, s, pl.ds(0, SEC)], sd0, isem0)
    plsc.subcore_barrier()

    bufs = (rows0, rows1)
    sems = (sem0, sem1)

    pltpu.make_async_copy(srcp_hbm.at[c, s, pl.ds(0, SEC)], sp0, isem0).wait()
    pltpu.make_async_copy(dst_hbm.at[c, s, pl.ds(0, SEC)], sd0, isem0).wait()
    pltpu.async_copy(g_hbm.at[sp0.at[0]], rows0, sem0)

    for sec in range(NSEC):
        par = sec % 2
        spb, sdb = sidx[par]
        if sec + 1 < NSEC:
            sl2 = pl.ds((sec + 1) * SEC, SEC)
            pltpu.async_copy(srcp_hbm.at[c, s, sl2], sidx[1 - par][0],
                             isems[1 - par])
            pltpu.async_copy(dst_hbm.at[c, s, sl2], sidx[1 - par][1],
                             isems[1 - par])

        def chunk(j, _, sec=sec, par=par, spb=spb, sdb=sdb):
            cur = lax.rem(j, 2)
            for b in range(2):
                @pl.when(cur == b)
                def _():
                    @pl.when(j + 1 < SEC)
                    def _():
                        pltpu.async_copy(g_hbm.at[spb.at[j + 1]],
                                         bufs[1 - b], sems[1 - b])
                    if sec + 1 < NSEC:
                        @pl.when(j + 1 == SEC)
                        def _():
                            nsl = pl.ds((sec + 1) * SEC, SEC)
                            pltpu.make_async_copy(
                                srcp_hbm.at[c, s, nsl], sidx[1 - par][0],
                                isems[1 - par]).wait()
                            pltpu.make_async_copy(
                                dst_hbm.at[c, s, nsl], sidx[1 - par][1],
                                isems[1 - par]).wait()
                            pltpu.async_copy(g_hbm.at[sidx[1 - par][0].at[0]],
                                             bufs[1 - b], sems[1 - b])
                    pltpu.make_async_copy(g_hbm.at[spb.at[j]], bufs[b],
                                          sems[b]).wait()
                    pltpu.sync_copy(bufs[b], acc.at[sdb.at[j]], add=True)
            return 0
        lax.fori_loop(0, SEC, chunk, 0)

    plsc.subcore_barrier()
    pltpu.sync_copy(acc.at[pl.ds(s * PSEG, PSEG)],
                    p_out.at[c, pl.ds(s * PSEG, PSEG)])


def _tc_call(body, out_shapes):
    return pl.pallas_call(
        body,
        out_shape=[jax.ShapeDtypeStruct(s, jnp.float32) for s in out_shapes],
    )


def _assemble(p_ref):
    return p_ref[0, :N, :] + p_ref[1, :N, :]


def _write_g(g_ref, gbody):
    g_ref[:N, :] = gbody
    g_ref[N:, :] = jnp.zeros((GP - N, D), jnp.float32)


def _prep_body(d0_ref, d1_ref, x_ref, dis_ref, g_ref):
    deg = d0_ref[...] + d1_ref[...]
    dis = jnp.where(deg > 0, lax.rsqrt(jnp.where(deg > 0, deg, 1.0)), 0.0)
    dis_ref[...] = dis
    _write_g(g_ref, dis * x_ref[...])


def _tca_body(tx0_ref, p_ref, dis_ref, w0_ref, w1_ref, g1_ref, oacc_ref):
    dis = dis_ref[...]
    tx1 = -(dis * _assemble(p_ref))
    oacc_ref[...] = (
        jnp.dot(tx0_ref[...], w0_ref[...], preferred_element_type=jnp.float32)
        + jnp.dot(tx1, w1_ref[...], preferred_element_type=jnp.float32))
    _write_g(g1_ref, dis * tx1)


def _tcb_body(tx0_ref, oacc_ref, q_ref, dis_ref, w2_ref, b_ref,
              ginv_ref, beta_ref, h_ref, gn_ref):
    dis = dis_ref[...]
    tx2 = -2.0 * (dis * _assemble(q_ref)) - tx0_ref[...]
    out = (oacc_ref[...]
           + jnp.dot(tx2, w2_ref[...], preferred_element_type=jnp.float32)
           + b_ref[...])
    h = jnp.maximum(out, 0.0) * ginv_ref[...] + beta_ref[...]
    h_ref[...] = h
    _write_g(gn_ref, dis * h)


def _final_body(h_ref, lw_ref, lb_ref, o_ref):
    o_ref[...] = jnp.dot(h_ref[...], lw_ref[...],
                         preferred_element_type=jnp.float32) + lb_ref[...]


def kernel(x, edge_index, W, b, gamma, beta, lin_W, lin_b):
    pad = (jnp.arange(EP - E, dtype=jnp.int32) * 13) % N
    src4 = jnp.concatenate([edge_index[0], pad]).reshape(NC, NS, HCH, CH)
    dst4 = jnp.concatenate([edge_index[1], pad]).reshape(NC, NS, HCH, CH)

    deg_flat, srcp4 = _sc_prep(src4, dst4)

    d0 = deg_flat[:N][:, None]
    d1 = deg_flat[NP:NP + N][:, None]
    dis_col, g = _tc_call(_prep_body, [(N, 1), (GP, D)])(d0, d1, x)

    ginv = gamma * (1.0 / jnp.sqrt(1.0 + EPS))
    lw_pad = jnp.zeros((D, D), jnp.float32).at[:, :OUT].set(lin_W)
    lb_pad = jnp.zeros((1, D), jnp.float32).at[0, :OUT].set(lin_b)

    wa = jnp.stack([W[:, 0], W[:, 2]], 1).reshape(2 * L, D, D)
    wb = jnp.stack([W[:, 1], W[:, 1]], 1).reshape(2 * L, D, D)
    zrow1 = jnp.zeros((L, 1, D), jnp.float32)
    brows = jnp.stack([zrow1, b[:, None, :]], 1).reshape(2 * L, 1, D)
    ginvrows = jnp.stack([zrow1, ginv[:, None, :]], 1).reshape(2 * L, 1, D)
    betarows = jnp.stack([zrow1, beta[:, None, :]], 1).reshape(2 * L, 1, D)
    flags = jnp.tile(jnp.array([1, 0], jnp.int32), L)

    def halfstep(carry, ws):
        h, g, oacc = carry
        w_a, w_b, brow, ginvrow, betarow, flag = ws
        p = _sc_prop(g, srcp4, dst4)

        def fa(h, g, oacc, p):
            g1, oacc1 = _tc_call(_tca_body, [(GP, D), (N, D)])(
                h, p, dis_col, w_a, w_b)
            return h, g1, oacc1

        def fb(h, g, oacc, p):
            h1, g1 = _tc_call(_tcb_body, [(N, D), (GP, D)])(
                h, oacc, p, dis_col, w_a, brow, ginvrow, betarow)
            return h1, g1, oacc

        h, g, oacc = lax.cond(flag == 1, fa, fb, h, g, oacc, p)
        return (h, g, oacc), None

    (h, _, _), _ = lax.scan(
        halfstep, (x, g, jnp.zeros((N, D), jnp.float32)),
        (wa, wb, brows, ginvrows, betarows, flags))

    logits_pad, = _tc_call(_final_body, [(N, D)])(h, lw_pad, lb_pad)
    return logits_pad[:, :OUT]


# speedup vs baseline: 1.1051x; 1.0088x over previous
import functools

import jax
import jax.numpy as jnp
from jax import lax
from jax.experimental import pallas as pl
from jax.experimental.pallas import tpu as pltpu
from jax.experimental.pallas import tpu_sc as plsc

N = 10000
E = 320000
D = 128
K = 3
L = 10
OUT = 10
EPS = 1e-5

NC = 2
NS = 16
CH = 64
EP = 327680
EW = EP // (NC * NS)
HCH = EW // CH
NBLK = HCH // 8
NP = 10240
DSEG = NP // NS
ZPAD = 128
GP = N + ZPAD
PR = 10112
PSEG = PR // NS
SEC = 32
NSEC = HCH // SEC

_sc_mesh = plsc.VectorSubcoreMesh(core_axis_name="c", subcore_axis_name="s")


@functools.partial(
    pl.kernel,
    name="sc_prep",
    out_type=(
        jax.ShapeDtypeStruct((NC * NP,), jnp.float32),
        jax.ShapeDtypeStruct((NC, NS, HCH, CH), jnp.int32),
    ),
    mesh=_sc_mesh,
    scratch_types=[
        pltpu.VMEM((8, CH), jnp.int32),
        pltpu.VMEM((8, CH), jnp.int32),
        pltpu.VMEM((8, CH), jnp.float32),
        pltpu.VMEM((DSEG,), jnp.float32),
        pltpu.VMEM_SHARED((NP,), jnp.float32),
    ],
)
def _sc_prep(src_hbm, dst_hbm, deg_out, srcp_out, src_v, dst_v, w_v, z_v,
             deg_acc):
    c = lax.axis_index("c")
    s = lax.axis_index("s")

    def zero16(i, _):
        z_v[pl.ds(i * 16, 16)] = jnp.zeros((16,), jnp.float32)
        return 0
    lax.fori_loop(0, DSEG // 16, zero16, 0)
    pltpu.sync_copy(z_v, deg_acc.at[pl.ds(s * DSEG, DSEG)])
    plsc.subcore_barrier()

    def block(ib, _):
        blk = pl.ds(ib * 8, 8)
        pltpu.sync_copy(src_hbm.at[c, s, blk], src_v)
        pltpu.sync_copy(dst_hbm.at[c, s, blk], dst_v)
        for i in range(8):
            for j in range(CH // 16):
                sl = pl.ds(j * 16, 16)
                s16 = src_v[i, sl]
                d16 = dst_v[i, sl]
                eq = s16 == d16
                w_v[i, sl] = jnp.where(eq, 0.0, 1.0).astype(jnp.float32)
                src_v[i, sl] = jnp.where(
                    eq, N + jnp.bitwise_and(s16, ZPAD - 1), s16)
        for i in range(8):
            pltpu.sync_copy(w_v.at[i], deg_acc.at[src_v.at[i]], add=True)
        pltpu.sync_copy(src_v, srcp_out.at[c, s, blk])
        return 0
    lax.fori_loop(0, NBLK, block, 0)

    plsc.subcore_barrier()
    pltpu.sync_copy(deg_acc.at[pl.ds(s * DSEG, DSEG)],
                    deg_out.at[pl.ds(c * NP + s * DSEG, DSEG)])


@functools.partial(
    pl.kernel,
    name="sc_prop",
    out_type=jax.ShapeDtypeStruct((NC, PR, D), jnp.float32),
    mesh=_sc_mesh,
    scratch_types=[
        pltpu.VMEM((SEC, CH), jnp.int32),
        pltpu.VMEM((SEC, CH), jnp.int32),
        pltpu.VMEM((SEC, CH), jnp.int32),
        pltpu.VMEM((SEC, CH), jnp.int32),
        pltpu.VMEM((CH, D), jnp.float32),
        pltpu.VMEM((CH, D), jnp.float32),
        pltpu.VMEM_SHARED((PR, D), jnp.float32),
        pltpu.SemaphoreType.DMA,
        pltpu.SemaphoreType.DMA,
        pltpu.SemaphoreType.DMA,
        pltpu.SemaphoreType.DMA,
    ],
)
def _sc_prop(g_hbm, srcp_hbm, dst_hbm, p_out,
             sp0, sp1, sd0, sd1, rows0, rows1, acc, sem0, sem1,
             isem0, isem1):
    c = lax.axis_index("c")
    s = lax.axis_index("s")

    def zrow_init(i, _):
        for j in range(D // 16):
            rows0[i, pl.ds(j * 16, 16)] = jnp.zeros((16,), jnp.float32)
        return 0
    lax.fori_loop(0, CH, zrow_init, 0)
    for t in range(PSEG // CH):
        pltpu.sync_copy(rows0, acc.at[pl.ds(s * PSEG + t * CH, CH)])
    pltpu.sync_copy(
        rows0.at[pl.ds(0, PSEG % CH)],
        acc.at[pl.ds(s * PSEG + (PSEG // CH) * CH, PSEG % CH)])

    sidx = ((sp0, sd0), (sp1, sd1))
    isems = (isem0, isem1)
    pltpu.async_copy(srcp_hbm.at[c, s, pl.ds(0, SEC)], sp0, isem0)
    pltpu.async_copy(dst_hbm.at[c, s, pl.ds(0, SEC)], sd0, isem0)
    plsc.subcore_barrier()

    bufs = (rows0, rows1)
    sems = (sem0, sem1)

    pltpu.make_async_copy(srcp_hbm.at[c, s, pl.ds(0, SEC)], sp0, isem0).wait()
    pltpu.make_async_copy(dst_hbm.at[c, s, pl.ds(0, SEC)], sd0, isem0).wait()
    pltpu.async_copy(g_hbm.at[sp0.at[0]], rows0, sem0)

    for sec in range(NSEC):
        par = sec % 2
        spb, sdb = sidx[par]
        if sec + 1 < NSEC:
            sl2 = pl.ds((sec + 1) * SEC, SEC)
            pltpu.async_copy(srcp_hbm.at[c, s, sl2], sidx[1 - par][0],
                             isems[1 - par])
            pltpu.async_copy(dst_hbm.at[c, s, sl2], sidx[1 - par][1],
                             isems[1 - par])

        def chunk(j, _, sec=sec, par=par, spb=spb, sdb=sdb):
            cur = lax.rem(j, 2)
            for b in range(2):
                @pl.when(cur == b)
                def _():
                    @pl.when(j + 1 < SEC)
                    def _():
                        pltpu.async_copy(g_hbm.at[spb.at[j + 1]],
                                         bufs[1 - b], sems[1 - b])
                    if sec + 1 < NSEC:
                        @pl.when(j + 1 == SEC)
                        def _():
                            nsl = pl.ds((sec + 1) * SEC, SEC)
                            pltpu.make_async_copy(
                                srcp_hbm.at[c, s, nsl], sidx[1 - par][0],
                                isems[1 - par]).wait()
                            pltpu.make_async_copy(
                                dst_hbm.at[c, s, nsl], sidx[1 - par][1],
                                isems[1 - par]).wait()
                            pltpu.async_copy(g_hbm.at[sidx[1 - par][0].at[0]],
                                             bufs[1 - b], sems[1 - b])
                    pltpu.make_async_copy(g_hbm.at[spb.at[j]], bufs[b],
                                          sems[b]).wait()
                    pltpu.sync_copy(bufs[b], acc.at[sdb.at[j]], add=True)
            return 0
        lax.fori_loop(0, SEC, chunk, 0)

    plsc.subcore_barrier()
    pltpu.sync_copy(acc.at[pl.ds(s * PSEG, PSEG)],
                    p_out.at[c, pl.ds(s * PSEG, PSEG)])


def _tc_call(body, out_shapes):
    return pl.pallas_call(
        body,
        out_shape=[jax.ShapeDtypeStruct(s, jnp.float32) for s in out_shapes],
    )


def _assemble(p_ref):
    return p_ref[0, :N, :] + p_ref[1, :N, :]


def _write_g(g_ref, gbody):
    g_ref[:N, :] = gbody
    g_ref[N:, :] = jnp.zeros((GP - N, D), jnp.float32)


def _prep_body(d0_ref, d1_ref, x_ref, dis_ref, g_ref):
    deg = d0_ref[...] + d1_ref[...]
    dis = jnp.where(deg > 0, lax.rsqrt(jnp.where(deg > 0, deg, 1.0)), 0.0)
    dis_ref[...] = dis
    _write_g(g_ref, dis * x_ref[...])


def _tca_body(tx0_ref, p_ref, dis_ref, w0_ref, w1_ref, g1_ref, oacc_ref):
    dis = dis_ref[...]
    tx1 = -(dis * _assemble(p_ref))
    oacc_ref[...] = (
        jnp.dot(tx0_ref[...], w0_ref[...], preferred_element_type=jnp.float32)
        + jnp.dot(tx1, w1_ref[...], preferred_element_type=jnp.float32))
    _write_g(g1_ref, dis * tx1)


def _tcb_body(tx0_ref, oacc_ref, q_ref, dis_ref, w2_ref, b_ref,
              ginv_ref, beta_ref, h_ref, gn_ref):
    dis = dis_ref[...]
    tx2 = -2.0 * (dis * _assemble(q_ref)) - tx0_ref[...]
    out = (oacc_ref[...]
           + jnp.dot(tx2, w2_ref[...], preferred_element_type=jnp.float32)
           + b_ref[...])
    h = jnp.maximum(out, 0.0) * ginv_ref[...] + beta_ref[...]
    h_ref[...] = h
    _write_g(gn_ref, dis * h)


def _final_body(h_ref, lw_ref, lb_ref, o_ref):
    o_ref[...] = jnp.dot(h_ref[...], lw_ref[...],
                         preferred_element_type=jnp.float32) + lb_ref[...]


def kernel(x, edge_index, W, b, gamma, beta, lin_W, lin_b):
    pad = (jnp.arange(EP - E, dtype=jnp.int32) * 13) % N
    src4 = jnp.concatenate([edge_index[0], pad]).reshape(NC, NS, HCH, CH)
    dst4 = jnp.concatenate([edge_index[1], pad]).reshape(NC, NS, HCH, CH)

    deg_flat, srcp4 = _sc_prep(src4, dst4)

    d0 = deg_flat[:N][:, None]
    d1 = deg_flat[NP:NP + N][:, None]
    dis_col, g = _tc_call(_prep_body, [(N, 1), (GP, D)])(d0, d1, x)

    ginv = gamma * (1.0 / jnp.sqrt(1.0 + EPS))
    lw_pad = jnp.zeros((D, D), jnp.float32).at[:, :OUT].set(lin_W)
    lb_pad = jnp.zeros((1, D), jnp.float32).at[0, :OUT].set(lin_b)

    wa = jnp.stack([W[:, 0], W[:, 2]], 1).reshape(2 * L, D, D)
    wb = jnp.stack([W[:, 1], W[:, 1]], 1).reshape(2 * L, D, D)
    zrow1 = jnp.zeros((L, 1, D), jnp.float32)
    brows = jnp.stack([zrow1, b[:, None, :]], 1).reshape(2 * L, 1, D)
    ginvrows = jnp.stack([zrow1, ginv[:, None, :]], 1).reshape(2 * L, 1, D)
    betarows = jnp.stack([zrow1, beta[:, None, :]], 1).reshape(2 * L, 1, D)
    flags = jnp.tile(jnp.array([1, 0], jnp.int32), L)

    def halfstep(carry, ws):
        h, g, oacc = carry
        w_a, w_b, brow, ginvrow, betarow, flag = ws
        p = _sc_prop(g, srcp4, dst4)

        def fa(h, g, oacc, p):
            g1, oacc1 = _tc_call(_tca_body, [(GP, D), (N, D)])(
                h, p, dis_col, w_a, w_b)
            return h, g1, oacc1

        def fb(h, g, oacc, p):
            h1, g1 = _tc_call(_tcb_body, [(N, D), (GP, D)])(
                h, oacc, p, dis_col, w_a, brow, ginvrow, betarow)
            return h1, g1, oacc

        h, g, oacc = lax.cond(flag == 1, fa, fb, h, g, oacc, p)
        return (h, g, oacc), None

    carry = (x, g, jnp.zeros((N, D), jnp.float32))
    for t in range(2 * L):
        carry, _ = halfstep(
            carry, (wa[t], wb[t], brows[t], ginvrows[t], betarows[t],
                    flags[t]))
    h = carry[0]

    logits_pad, = _tc_call(_final_body, [(N, D)])(h, lw_pad, lb_pad)
    return logits_pad[:, :OUT]


# speedup vs baseline: 1.1295x; 1.0221x over previous
import functools

import jax
import jax.numpy as jnp
from jax import lax
from jax.experimental import pallas as pl
from jax.experimental.pallas import tpu as pltpu
from jax.experimental.pallas import tpu_sc as plsc

N = 10000
E = 320000
D = 128
K = 3
L = 10
OUT = 10
EPS = 1e-5

NC = 2
NS = 16
CH = 64
EP = 327680
EW = EP // (NC * NS)
HCH = EW // CH
NBLK = HCH // 8
NP = 10240
DSEG = NP // NS
ZPAD = 128
GP = N + ZPAD
PR = 10112
PSEG = PR // NS
SEC = 32
NSEC = HCH // SEC

_sc_mesh = plsc.VectorSubcoreMesh(core_axis_name="c", subcore_axis_name="s")


@functools.partial(
    pl.kernel,
    name="sc_prep",
    out_type=(
        jax.ShapeDtypeStruct((NC * NP,), jnp.float32),
        jax.ShapeDtypeStruct((NC, NS, HCH, CH), jnp.int32),
    ),
    mesh=_sc_mesh,
    scratch_types=[
        pltpu.VMEM((8, CH), jnp.int32),
        pltpu.VMEM((8, CH), jnp.int32),
        pltpu.VMEM((8, CH), jnp.float32),
        pltpu.VMEM((DSEG,), jnp.float32),
        pltpu.VMEM_SHARED((NP,), jnp.float32),
    ],
)
def _sc_prep(src_hbm, dst_hbm, deg_out, srcp_out, src_v, dst_v, w_v, z_v,
             deg_acc):
    c = lax.axis_index("c")
    s = lax.axis_index("s")

    def zero16(i, _):
        z_v[pl.ds(i * 16, 16)] = jnp.zeros((16,), jnp.float32)
        return 0
    lax.fori_loop(0, DSEG // 16, zero16, 0)
    pltpu.sync_copy(z_v, deg_acc.at[pl.ds(s * DSEG, DSEG)])
    plsc.subcore_barrier()

    def block(ib, _):
        blk = pl.ds(ib * 8, 8)
        pltpu.sync_copy(src_hbm.at[c, s, blk], src_v)
        pltpu.sync_copy(dst_hbm.at[c, s, blk], dst_v)
        for i in range(8):
            for j in range(CH // 16):
                sl = pl.ds(j * 16, 16)
                s16 = src_v[i, sl]
                d16 = dst_v[i, sl]
                eq = s16 == d16
                w_v[i, sl] = jnp.where(eq, 0.0, 1.0).astype(jnp.float32)
                src_v[i, sl] = jnp.where(
                    eq, N + jnp.bitwise_and(s16, ZPAD - 1), s16)
        for i in range(8):
            pltpu.sync_copy(w_v.at[i], deg_acc.at[src_v.at[i]], add=True)
        pltpu.sync_copy(src_v, srcp_out.at[c, s, blk])
        return 0
    lax.fori_loop(0, NBLK, block, 0)

    plsc.subcore_barrier()
    pltpu.sync_copy(deg_acc.at[pl.ds(s * DSEG, DSEG)],
                    deg_out.at[pl.ds(c * NP + s * DSEG, DSEG)])


@functools.partial(
    pl.kernel,
    name="sc_prop",
    out_type=jax.ShapeDtypeStruct((NC, PR, D), jnp.float32),
    mesh=_sc_mesh,
    scratch_types=[
        pltpu.VMEM((SEC, CH), jnp.int32),
        pltpu.VMEM((SEC, CH), jnp.int32),
        pltpu.VMEM((SEC, CH), jnp.int32),
        pltpu.VMEM((SEC, CH), jnp.int32),
        pltpu.VMEM((CH, D), jnp.float32),
        pltpu.VMEM((CH, D), jnp.float32),
        pltpu.VMEM_SHARED((PR, D), jnp.float32),
        pltpu.SemaphoreType.DMA,
        pltpu.SemaphoreType.DMA,
        pltpu.SemaphoreType.DMA,
        pltpu.SemaphoreType.DMA,
    ],
)
def _sc_prop(g_hbm, srcp_hbm, dst_hbm, p_out,
             sp0, sp1, sd0, sd1, rows0, rows1, acc, sem0, sem1,
             isem0, isem1):
    c = lax.axis_index("c")
    s = lax.axis_index("s")

    def zrow_init(i, _):
        for j in range(D // 16):
            rows0[i, pl.ds(j * 16, 16)] = jnp.zeros((16,), jnp.float32)
        return 0
    lax.fori_loop(0, CH, zrow_init, 0)
    for t in range(PSEG // CH):
        pltpu.sync_copy(rows0, acc.at[pl.ds(s * PSEG + t * CH, CH)])
    pltpu.sync_copy(
        rows0.at[pl.ds(0, PSEG % CH)],
        acc.at[pl.ds(s * PSEG + (PSEG // CH) * CH, PSEG % CH)])

    sidx = ((sp0, sd0), (sp1, sd1))
    isems = (isem0, isem1)
    pltpu.async_copy(srcp_hbm.at[c, s, pl.ds(0, SEC)], sp0, isem0)
    pltpu.async_copy(dst_hbm.at[c, s, pl.ds(0, SEC)], sd0, isem0)
    plsc.subcore_barrier()

    bufs = (rows0, rows1)
    sems = (sem0, sem1)

    pltpu.make_async_copy(srcp_hbm.at[c, s, pl.ds(0, SEC)], sp0, isem0).wait()
    pltpu.make_async_copy(dst_hbm.at[c, s, pl.ds(0, SEC)], sd0, isem0).wait()
    pltpu.async_copy(g_hbm.at[sp0.at[0]], rows0, sem0)

    for sec in range(NSEC):
        par = sec % 2
        spb, sdb = sidx[par]
        if sec + 1 < NSEC:
            sl2 = pl.ds((sec + 1) * SEC, SEC)
            pltpu.async_copy(srcp_hbm.at[c, s, sl2], sidx[1 - par][0],
                             isems[1 - par])
            pltpu.async_copy(dst_hbm.at[c, s, sl2], sidx[1 - par][1],
                             isems[1 - par])

        def chunk(j, _, sec=sec, par=par, spb=spb, sdb=sdb):
            cur = lax.rem(j, 2)
            for b in range(2):
                @pl.when(cur == b)
                def _():
                    @pl.when(j + 1 < SEC)
                    def _():
                        pltpu.async_copy(g_hbm.at[spb.at[j + 1]],
                                         bufs[1 - b], sems[1 - b])
                    if sec + 1 < NSEC:
                        @pl.when(j + 1 == SEC)
                        def _():
                            nsl = pl.ds((sec + 1) * SEC, SEC)
                            pltpu.make_async_copy(
                                srcp_hbm.at[c, s, nsl], sidx[1 - par][0],
                                isems[1 - par]).wait()
                            pltpu.make_async_copy(
                                dst_hbm.at[c, s, nsl], sidx[1 - par][1],
                                isems[1 - par]).wait()
                            pltpu.async_copy(g_hbm.at[sidx[1 - par][0].at[0]],
                                             bufs[1 - b], sems[1 - b])
                    pltpu.make_async_copy(g_hbm.at[spb.at[j]], bufs[b],
                                          sems[b]).wait()
                    pltpu.sync_copy(bufs[b], acc.at[sdb.at[j]], add=True)
            return 0
        lax.fori_loop(0, SEC, chunk, 0)

    plsc.subcore_barrier()
    pltpu.sync_copy(acc.at[pl.ds(s * PSEG, PSEG)],
                    p_out.at[c, pl.ds(s * PSEG, PSEG)])


def _tc_call(body, out_shapes):
    return pl.pallas_call(
        body,
        out_shape=[jax.ShapeDtypeStruct(s, jnp.float32) for s in out_shapes],
    )


def _assemble(p_ref):
    return p_ref[0, :N, :] + p_ref[1, :N, :]


def _write_g(g_ref, gbody):
    g_ref[:N, :] = gbody
    g_ref[N:, :] = jnp.zeros((GP - N, D), jnp.float32)


def _prep_body(d0_ref, d1_ref, x_ref, dis_ref, g_ref):
    deg = d0_ref[...] + d1_ref[...]
    dis = jnp.where(deg > 0, lax.rsqrt(jnp.where(deg > 0, deg, 1.0)), 0.0)
    dis_ref[...] = dis
    _write_g(g_ref, dis * x_ref[...])


def _tca_body(tx0_ref, p_ref, dis_ref, w0_ref, w1_ref, g1_ref, oacc_ref):
    dis = dis_ref[...]
    tx1 = -(dis * _assemble(p_ref))
    oacc_ref[...] = (
        jnp.dot(tx0_ref[...], w0_ref[...], preferred_element_type=jnp.float32)
        + jnp.dot(tx1, w1_ref[...], preferred_element_type=jnp.float32))
    _write_g(g1_ref, dis * tx1)


def _tcb_body(tx0_ref, oacc_ref, q_ref, dis_ref, w2_ref, b_ref,
              ginv_ref, beta_ref, h_ref, gn_ref):
    dis = dis_ref[...]
    tx2 = -2.0 * (dis * _assemble(q_ref)) - tx0_ref[...]
    out = (oacc_ref[...]
           + jnp.dot(tx2, w2_ref[...], preferred_element_type=jnp.float32)
           + b_ref[...])
    h = jnp.maximum(out, 0.0) * ginv_ref[...] + beta_ref[...]
    h_ref[...] = h
    _write_g(gn_ref, dis * h)


def _final_body(h_ref, lw_ref, lb_ref, o_ref):
    o_ref[...] = jnp.dot(h_ref[...], lw_ref[...],
                         preferred_element_type=jnp.float32) + lb_ref[...]


def kernel(x, edge_index, W, b, gamma, beta, lin_W, lin_b):
    pad = (jnp.arange(EP - E, dtype=jnp.int32) * 13) % N
    src4 = jnp.concatenate([edge_index[0], pad]).reshape(NC, NS, HCH, CH)
    dst4 = jnp.concatenate([edge_index[1], pad]).reshape(NC, NS, HCH, CH)

    deg_flat, srcp4 = _sc_prep(src4, dst4)

    d0 = deg_flat[:N][:, None]
    d1 = deg_flat[NP:NP + N][:, None]
    dis_col, g = _tc_call(_prep_body, [(N, 1), (GP, D)])(d0, d1, x)

    ginv = gamma * (1.0 / jnp.sqrt(1.0 + EPS))
    lw_pad = jnp.zeros((D, D), jnp.float32).at[:, :OUT].set(lin_W)
    lb_pad = jnp.zeros((1, D), jnp.float32).at[0, :OUT].set(lin_b)

    wa = jnp.stack([W[:, 0], W[:, 2]], 1).reshape(2 * L, D, D)
    wb = jnp.stack([W[:, 1], W[:, 1]], 1).reshape(2 * L, D, D)
    zrow1 = jnp.zeros((L, 1, D), jnp.float32)
    brows = jnp.stack([zrow1, b[:, None, :]], 1).reshape(2 * L, 1, D)
    ginvrows = jnp.stack([zrow1, ginv[:, None, :]], 1).reshape(2 * L, 1, D)
    betarows = jnp.stack([zrow1, beta[:, None, :]], 1).reshape(2 * L, 1, D)
    flags = jnp.tile(jnp.array([1, 0], jnp.int32), L)

    h = x
    oacc = None
    for t in range(2 * L):
        p = _sc_prop(g, srcp4, dst4)
        if t % 2 == 0:
            g, oacc = _tc_call(_tca_body, [(GP, D), (N, D)])(
                h, p, dis_col, wa[t], wb[t])
        else:
            h, g = _tc_call(_tcb_body, [(N, D), (GP, D)])(
                h, oacc, p, dis_col, wa[t], brows[t], ginvrows[t],
                betarows[t])

    logits_pad, = _tc_call(_final_body, [(N, D)])(h, lw_pad, lb_pad)
    return logits_pad[:, :OUT]
